# baseline (speedup 1.0000x reference)
import functools

import jax
import jax.numpy as jnp
from jax import lax
from jax.experimental import pallas as pl
from jax.experimental.pallas import tpu as pltpu
from jax.experimental.pallas import tpu_sc as plsc

B = 4096
NF = 26
V = 100000
D = 16
H = 50

NC = 2
NS = 16
NW = NC * NS
L = 16

S_TOT = B * NF
V_TOT = B * H
S_PER = S_TOT // NW
V_PER = V_TOT // NW

CH = 256
S_CHUNKS = S_PER // CH
V_CHUNKS = V_PER // CH

_mesh = plsc.VectorSubcoreMesh(
    core_axis_name="c", subcore_axis_name="s", num_cores=NC, num_subcores=NS
)


@functools.partial(
    pl.kernel,
    out_type=(
        jax.ShapeDtypeStruct((S_TOT * D,), jnp.float32),
        jax.ShapeDtypeStruct((V_TOT * D,), jnp.float32),
    ),
    mesh=_mesh,
    compiler_params=pltpu.CompilerParams(needs_layout_passes=False),
    scratch_types=[
        pltpu.VMEM((S_PER,), jnp.int32),
        pltpu.VMEM((S_PER,), jnp.int32),
        pltpu.VMEM((V_PER,), jnp.int32),
        pltpu.VMEM((V_PER,), jnp.int32),
        pltpu.VMEM((CH, 128), jnp.float32),
        pltpu.VMEM((CH * D,), jnp.float32),
        pltpu.SemaphoreType.DMA,
    ],
)
def _gather_all(s_idx_hbm, v_idx_hbm, tbl_hbm, vtbl_hbm, s_out, v_out,
                sgrp, soff, vgrp, voff, gath, outb, sem):
    wid = lax.axis_index("s") * NC + lax.axis_index("c")
    sbase = wid * S_PER
    vbase = wid * V_PER
    lane = lax.iota(jnp.int32, L)

    pltpu.sync_copy(s_idx_hbm.at[pl.ds(sbase, S_PER)], sgrp)
    pltpu.sync_copy(v_idx_hbm.at[pl.ds(vbase, V_PER)], vgrp)

    def sprep(j, carry):
        o = j * L
        pos = (sbase + o) + lane
        flat = sgrp[pl.ds(o, L)] + (pos % NF) * V
        sgrp[pl.ds(o, L)] = flat >> 3
        soff[pl.ds(o, L)] = (flat & 7) * D
        return carry

    lax.fori_loop(0, S_PER // L, sprep, 0, unroll=4)

    def vprep(j, carry):
        o = j * L
        raw = vgrp[pl.ds(o, L)]
        vgrp[pl.ds(o, L)] = raw >> 3
        voff[pl.ds(o, L)] = (raw & 7) * D
        return carry

    lax.fori_loop(0, V_PER // L, vprep, 0, unroll=4)

    def make_phase(grp, off_ref, tbl, out, obase):
        def chunk(c, carry):
            cb = c * CH
            pltpu.async_copy(tbl.at[grp.at[pl.ds(cb, CH)]], gath, sem).wait()

            def block(b, carry2):
                r0 = b * L
                offs = off_ref[pl.ds(cb + r0, L)]
                rows = r0 + lane
                opos = rows * D
                for j in range(D):
                    v = plsc.load_gather(gath, [rows, offs + j])
                    plsc.store_scatter(outb, [opos + j], v)
                return carry2

            lax.fori_loop(0, CH // L, block, 0)
            pltpu.sync_copy(outb, out.at[pl.ds(obase + cb * D, CH * D)])
            return carry

        return chunk

    lax.fori_loop(0, S_CHUNKS, make_phase(sgrp, soff, tbl_hbm, s_out, sbase * D), 0)
    lax.fori_loop(0, V_CHUNKS, make_phase(vgrp, voff, vtbl_hbm, v_out, vbase * D), 0)


def kernel(sparse_idx, varlen_idx, dense_vals, sparse_tables, varlen_table):
    s_out, v_out = _gather_all(
        sparse_idx.reshape(S_TOT),
        varlen_idx.reshape(V_TOT),
        sparse_tables.reshape(NF * V // 8, 128),
        varlen_table.reshape(V // 8, 128),
    )
    return s_out.reshape(B, NF, D), v_out.reshape(B, H, D), dense_vals


# speedup vs baseline: 6.3974x; 6.3974x over previous
import functools

import jax
import jax.numpy as jnp
from jax import lax
from jax.experimental import pallas as pl
from jax.experimental.pallas import tpu as pltpu
from jax.experimental.pallas import tpu_sc as plsc

B = 4096
NF = 26
V = 100000
D = 16
H = 50

NC = 2
NS = 16
NW = NC * NS
L = 16

S_UNITS = NF * D
V_UNITS = H * D
S_U_PER = S_UNITS // NW
V_U_PER = V_UNITS // NW

_mesh = plsc.VectorSubcoreMesh(
    core_axis_name="c", subcore_axis_name="s", num_cores=NC, num_subcores=NS
)


@functools.partial(
    pl.kernel,
    out_type=(
        jax.ShapeDtypeStruct((NF, D, B), jnp.float32),
        jax.ShapeDtypeStruct((H, D, B), jnp.float32),
    ),
    mesh=_mesh,
    compiler_params=pltpu.CompilerParams(needs_layout_passes=False),
    scratch_types=[
        pltpu.VMEM((V,), jnp.float32),
        pltpu.VMEM((B,), jnp.int32),
        pltpu.VMEM((B,), jnp.float32),
    ],
)
def _emb(s_idx_t, v_idx_t, tbl_t, vtbl_t, s_out, v_out, row_v, idx_v, outb):
    wid = lax.axis_index("s") * NC + lax.axis_index("c")

    def gather_row():
        def blk(i, c):
            ids = idx_v[pl.ds(i * L, L)]
            outb[pl.ds(i * L, L)] = plsc.load_gather(row_v, [ids])
            return c

        lax.fori_loop(0, B // L, blk, 0, unroll=4)

    for k in range(S_U_PER):
        u = wid * S_U_PER + k
        f = u // D
        d = u % D
        pltpu.sync_copy(s_idx_t.at[f], idx_v)
        pltpu.sync_copy(tbl_t.at[f, d], row_v)
        gather_row()
        pltpu.sync_copy(outb, s_out.at[f, d])

    dv = wid // 2
    pltpu.sync_copy(vtbl_t.at[dv], row_v)
    for k in range(V_U_PER):
        h = (wid % 2) * V_U_PER + k
        pltpu.sync_copy(v_idx_t.at[h], idx_v)
        gather_row()
        pltpu.sync_copy(outb, v_out.at[h, dv])


def kernel(sparse_idx, varlen_idx, dense_vals, sparse_tables, varlen_table):
    s_out, v_out = _emb(
        sparse_idx.T,
        varlen_idx.T,
        jnp.transpose(sparse_tables, (0, 2, 1)),
        varlen_table.T,
    )
    return (
        jnp.transpose(s_out, (2, 0, 1)),
        jnp.transpose(v_out, (2, 0, 1)),
        dense_vals,
    )


# speedup vs baseline: 7.3415x; 1.1476x over previous
import functools

import jax
import jax.numpy as jnp
from jax import lax
from jax.experimental import pallas as pl
from jax.experimental.pallas import tpu as pltpu
from jax.experimental.pallas import tpu_sc as plsc

B = 4096
NF = 26
V = 100000
D = 16
H = 50

NC = 2
NS = 16
NW = NC * NS
L = 16

S_UNITS = NF * D
V_UNITS = H * D
S_U_PER = S_UNITS // NW
V_U_PER = V_UNITS // NW

_mesh = plsc.VectorSubcoreMesh(
    core_axis_name="c", subcore_axis_name="s", num_cores=NC, num_subcores=NS
)


@functools.partial(
    pl.kernel,
    out_type=(
        jax.ShapeDtypeStruct((NF, D, B), jnp.float32),
        jax.ShapeDtypeStruct((H, D, B), jnp.float32),
    ),
    mesh=_mesh,
    compiler_params=pltpu.CompilerParams(needs_layout_passes=False),
    scratch_types=[
        pltpu.VMEM((V,), jnp.float32),
        pltpu.VMEM((B,), jnp.int32),
        pltpu.VMEM((B,), jnp.int32),
        pltpu.VMEM((B,), jnp.float32),
        pltpu.VMEM((B,), jnp.float32),
        pltpu.SemaphoreType.DMA,
        pltpu.SemaphoreType.DMA,
        pltpu.SemaphoreType.DMA,
        pltpu.SemaphoreType.DMA,
    ],
)
def _emb(s_idx_t, v_idx_t, tbl_t, vtbl_t, s_out, v_out,
         row_v, idx0, idx1, outb0, outb1, so0, so1, si0, si1):
    wid = lax.axis_index("s") * NC + lax.axis_index("c")
    outbs = (outb0, outb1)
    osems = (so0, so1)
    idxs = (idx0, idx1)
    isems = (si0, si1)

    def gather_row(idx_v, outb):
        def blk(i, c):
            ids = idx_v[pl.ds(i * L, L)]
            outb[pl.ds(i * L, L)] = plsc.load_gather(row_v, [ids])
            return c

        lax.fori_loop(0, B // L, blk, 0, unroll=8)

    pending = [None, None]
    for k in range(S_U_PER):
        u = wid * S_U_PER + k
        f = u // D
        d = u % D
        ob = outbs[k % 2]
        pltpu.sync_copy(s_idx_t.at[f], idx0)
        pltpu.sync_copy(tbl_t.at[f, d], row_v)
        if pending[k % 2] is not None:
            pending[k % 2].wait()
        gather_row(idx0, ob)
        pending[k % 2] = pltpu.async_copy(ob, s_out.at[f, d], osems[k % 2])

    dv = wid // 2
    h0 = (wid % 2) * V_U_PER
    ipend = [None, None]
    ipend[0] = pltpu.async_copy(v_idx_t.at[h0], idxs[0], isems[0])
    pltpu.sync_copy(vtbl_t.at[dv], row_v)
    for k in range(V_U_PER):
        h = h0 + k
        ob = outbs[k % 2]
        ipend[k % 2].wait()
        if k + 1 < V_U_PER:
            ipend[(k + 1) % 2] = pltpu.async_copy(
                v_idx_t.at[h + 1], idxs[(k + 1) % 2], isems[(k + 1) % 2]
            )
        if pending[k % 2] is not None:
            pending[k % 2].wait()
        gather_row(idxs[k % 2], ob)
        pending[k % 2] = pltpu.async_copy(ob, v_out.at[h, dv], osems[k % 2])
    pending[0].wait()
    pending[1].wait()


def kernel(sparse_idx, varlen_idx, dense_vals, sparse_tables, varlen_table):
    s_out, v_out = _emb(
        sparse_idx.T,
        varlen_idx.T,
        jnp.transpose(sparse_tables, (0, 2, 1)),
        varlen_table.T,
    )
    return (
        jnp.transpose(s_out, (2, 0, 1)),
        jnp.transpose(v_out, (2, 0, 1)),
        dense_vals,
    )


# speedup vs baseline: 7.5874x; 1.0335x over previous
import functools

import jax
import jax.numpy as jnp
from jax import lax
from jax.experimental import pallas as pl
from jax.experimental.pallas import tpu as pltpu
from jax.experimental.pallas import tpu_sc as plsc

B = 4096
NF = 26
V = 100000
D = 16
H = 50

NC = 2
NS = 16
NW = NC * NS
L = 16

S_UNITS = NF * D
V_UNITS = H * D
S_U_PER = S_UNITS // NW
V_U_PER = V_UNITS // NW

_mesh = plsc.VectorSubcoreMesh(
    core_axis_name="c", subcore_axis_name="s", num_cores=NC, num_subcores=NS
)


@functools.partial(
    pl.kernel,
    out_type=(
        jax.ShapeDtypeStruct((NF, D, B), jnp.float32),
        jax.ShapeDtypeStruct((H, D, B), jnp.float32),
    ),
    mesh=_mesh,
    compiler_params=pltpu.CompilerParams(needs_layout_passes=False),
    scratch_types=[
        pltpu.VMEM((V,), jnp.float32),
        pltpu.VMEM((B,), jnp.int32),
        pltpu.VMEM((B,), jnp.int32),
        pltpu.VMEM((B,), jnp.float32),
        pltpu.VMEM((B,), jnp.float32),
        pltpu.SemaphoreType.DMA,
        pltpu.SemaphoreType.DMA,
        pltpu.SemaphoreType.DMA,
        pltpu.SemaphoreType.DMA,
        pltpu.SemaphoreType.DMA,
    ],
)
def _emb(s_idx_t, v_idx_t, tbl_t, vtbl_t, s_out, v_out,
         row_v, idx0, idx1, outb0, outb1, so0, so1, si0, si1, srow):
    wid = lax.axis_index("s") * NC + lax.axis_index("c")
    outbs = (outb0, outb1)
    osems = (so0, so1)
    idxs = (idx0, idx1)
    isems = (si0, si1)

    def gather_row(idx_v, outb):
        def blk(i, c):
            ids = idx_v[pl.ds(i * L, L)]
            outb[pl.ds(i * L, L)] = plsc.load_gather(row_v, [ids])
            return c

        lax.fori_loop(0, B // L, blk, 0, unroll=16)

    def sfd(k):
        u = wid * S_U_PER + k
        return u // D, u % D

    pending = [None, None]
    ipend = [None, None]
    f0, d0 = sfd(0)
    rpend = pltpu.async_copy(tbl_t.at[f0, d0], row_v, srow)
    ipend[0] = pltpu.async_copy(s_idx_t.at[f0], idxs[0], isems[0])
    for k in range(S_U_PER):
        f, d = sfd(k)
        ob = outbs[k % 2]
        ipend[k % 2].wait()
        if k + 1 < S_U_PER:
            fn, _ = sfd(k + 1)
            ipend[(k + 1) % 2] = pltpu.async_copy(
                s_idx_t.at[fn], idxs[(k + 1) % 2], isems[(k + 1) % 2]
            )
        if pending[k % 2] is not None:
            pending[k % 2].wait()
        rpend.wait()
        gather_row(idxs[k % 2], ob)
        if k + 1 < S_U_PER:
            fn, dn = sfd(k + 1)
            rpend = pltpu.async_copy(tbl_t.at[fn, dn], row_v, srow)
        pending[k % 2] = pltpu.async_copy(ob, s_out.at[f, d], osems[k % 2])

    dv = wid // 2
    h0 = (wid % 2) * V_U_PER
    ipend = [None, None]
    ipend[0] = pltpu.async_copy(v_idx_t.at[h0], idxs[0], isems[0])
    pltpu.sync_copy(vtbl_t.at[dv], row_v)
    for k in range(V_U_PER):
        h = h0 + k
        ob = outbs[k % 2]
        ipend[k % 2].wait()
        if k + 1 < V_U_PER:
            ipend[(k + 1) % 2] = pltpu.async_copy(
                v_idx_t.at[h + 1], idxs[(k + 1) % 2], isems[(k + 1) % 2]
            )
        if pending[k % 2] is not None:
            pending[k % 2].wait()
        gather_row(idxs[k % 2], ob)
        pending[k % 2] = pltpu.async_copy(ob, v_out.at[h, dv], osems[k % 2])
    pending[0].wait()
    pending[1].wait()


def kernel(sparse_idx, varlen_idx, dense_vals, sparse_tables, varlen_table):
    s_out, v_out = _emb(
        sparse_idx.T,
        varlen_idx.T,
        jnp.transpose(sparse_tables, (0, 2, 1)),
        varlen_table.T,
    )
    return (
        jnp.transpose(s_out, (2, 0, 1)),
        jnp.transpose(v_out, (2, 0, 1)),
        dense_vals,
    )


# speedup vs baseline: 7.6028x; 1.0020x over previous
import functools

import jax
import jax.numpy as jnp
from jax import lax
from jax.experimental import pallas as pl
from jax.experimental.pallas import tpu as pltpu
from jax.experimental.pallas import tpu_sc as plsc

B = 4096
NF = 26
V = 100000
D = 16
H = 50

NC = 2
NS = 16
NW = NC * NS
L = 16

S_UNITS = NF * D
V_UNITS = H * D
S_U_PER = S_UNITS // NW
V_U_PER = V_UNITS // NW

_mesh = plsc.VectorSubcoreMesh(
    core_axis_name="c", subcore_axis_name="s", num_cores=NC, num_subcores=NS
)


@functools.partial(
    pl.kernel,
    out_type=(
        jax.ShapeDtypeStruct((NF, D, B), jnp.float32),
        jax.ShapeDtypeStruct((H, D, B), jnp.float32),
    ),
    mesh=_mesh,
    compiler_params=pltpu.CompilerParams(needs_layout_passes=False),
    scratch_types=[
        pltpu.VMEM((V,), jnp.float32),
        pltpu.VMEM((B,), jnp.int32),
        pltpu.VMEM((B,), jnp.int32),
        pltpu.VMEM((B,), jnp.float32),
        pltpu.VMEM((B,), jnp.float32),
        pltpu.SemaphoreType.DMA,
        pltpu.SemaphoreType.DMA,
        pltpu.SemaphoreType.DMA,
        pltpu.SemaphoreType.DMA,
        pltpu.SemaphoreType.DMA,
    ],
)
def _emb(s_idx_t, v_idx_t, tbl_t, vtbl_t, s_out, v_out,
         row_v, idx0, idx1, outb0, outb1, so0, so1, si0, si1, srow):
    wid = lax.axis_index("s") * NC + lax.axis_index("c")
    outbs = (outb0, outb1)
    osems = (so0, so1)
    idxs = (idx0, idx1)
    isems = (si0, si1)

    def gather_row(idx_v, outb):
        def blk(i, c):
            ids = idx_v[pl.ds(i * L, L)]
            outb[pl.ds(i * L, L)] = plsc.load_gather(row_v, [ids])
            return c

        lax.fori_loop(0, B // L, blk, 0, unroll=8)

    def sfd(k):
        u = wid * S_U_PER + k
        return u // D, u % D

    def sparse_phase():
        pending = [None, None]
        ipend = [None, None]
        f0, d0 = sfd(0)
        rpend = pltpu.async_copy(tbl_t.at[f0, d0], row_v, srow)
        ipend[0] = pltpu.async_copy(s_idx_t.at[f0], idxs[0], isems[0])
        for k in range(S_U_PER):
            f, d = sfd(k)
            ob = outbs[k % 2]
            ipend[k % 2].wait()
            if k + 1 < S_U_PER:
                fn, _ = sfd(k + 1)
                ipend[(k + 1) % 2] = pltpu.async_copy(
                    s_idx_t.at[fn], idxs[(k + 1) % 2], isems[(k + 1) % 2]
                )
            if pending[k % 2] is not None:
                pending[k % 2].wait()
            rpend.wait()
            gather_row(idxs[k % 2], ob)
            if k + 1 < S_U_PER:
                fn, dn = sfd(k + 1)
                rpend = pltpu.async_copy(tbl_t.at[fn, dn], row_v, srow)
            pending[k % 2] = pltpu.async_copy(ob, s_out.at[f, d], osems[k % 2])
        pending[(S_U_PER - 1) % 2].wait()
        pending[S_U_PER % 2].wait()

    def varlen_phase():
        pending = [None, None]
        dv = wid // 2
        h0 = (wid % 2) * V_U_PER
        ipend = [None, None]
        ipend[0] = pltpu.async_copy(v_idx_t.at[h0], idxs[0], isems[0])
        pltpu.sync_copy(vtbl_t.at[dv], row_v)
        for k in range(V_U_PER):
            h = h0 + k
            ob = outbs[k % 2]
            ipend[k % 2].wait()
            if k + 1 < V_U_PER:
                ipend[(k + 1) % 2] = pltpu.async_copy(
                    v_idx_t.at[h + 1], idxs[(k + 1) % 2], isems[(k + 1) % 2]
                )
            if pending[k % 2] is not None:
                pending[k % 2].wait()
            gather_row(idxs[k % 2], ob)
            pending[k % 2] = pltpu.async_copy(ob, v_out.at[h, dv], osems[k % 2])
        pending[(V_U_PER - 1) % 2].wait()
        pending[V_U_PER % 2].wait()

    even = (wid % 2) == 0

    @pl.when(even)
    def _():
        sparse_phase()
        varlen_phase()

    @pl.when(jnp.logical_not(even))
    def _():
        varlen_phase()
        sparse_phase()


def kernel(sparse_idx, varlen_idx, dense_vals, sparse_tables, varlen_table):
    s_out, v_out = _emb(
        sparse_idx.T,
        varlen_idx.T,
        jnp.transpose(sparse_tables, (0, 2, 1)),
        varlen_table.T,
    )
    return (
        jnp.transpose(s_out, (2, 0, 1)),
        jnp.transpose(v_out, (2, 0, 1)),
        dense_vals,
    )
